# baseline (speedup 1.0000x reference)
import functools

import jax
import jax.numpy as jnp
from jax import lax
from jax.experimental import pallas as pl
from jax.experimental.pallas import tpu as pltpu
from jax.experimental.pallas import tpu_sc as plsc

N = 100000
E = 3200000
W = 32
NP = 102400
SPW = NP // W
CHW = 100096
ECL = E - 31 * CHW
BLK = 4096
EPAD = E + BLK
CSP = E + 512
WALK_LEN = 3

_i32 = jnp.int32
_f32 = jnp.float32


def _wid():
    return (lax.axis_index("s") * _i32(2) + lax.axis_index("c")).astype(_i32)


def _iota16():
    return lax.iota(_i32, 16)


def _edge_count(w):
    return jnp.where(w == _i32(W - 1), _i32(ECL), _i32(CHW)).astype(_i32)


def _k1_hist(row_hbm, hist_hbm, histv, rowb, sem):
    w = _wid()

    def zero(j, c):
        histv[pl.ds(j * _i32(16), 16)] = jnp.zeros((16,), _i32)
        return c

    lax.fori_loop(_i32(0), _i32(NP // 16), zero, _i32(0))
    start = w * _i32(CHW)
    ec = _edge_count(w)
    nblk = (ec + _i32(BLK - 1)) // _i32(BLK)

    def blk(b, c):
        pltpu.sync_copy(row_hbm.at[pl.ds(start + b * _i32(BLK), BLK)], rowb)
        bs = jnp.minimum(_i32(BLK), ec - b * _i32(BLK))

        def vreg(j, c2):
            r = rowb[pl.ds(j * _i32(16), 16)]
            cnt, last = plsc.scan_count(r)
            plsc.addupdate_scatter(histv, [r], cnt, mask=last)
            return c2

        lax.fori_loop(_i32(0), bs // _i32(16), vreg, c)
        return c

    lax.fori_loop(_i32(0), nblk, blk, _i32(0))
    pltpu.sync_copy(histv, hist_hbm.at[w])


def _k2_prefix(hist_hbm, p_hbm, deg_hbm, sums_hbm, accv, tmpv, sumv, sem):
    w = _wid()
    cb = w * _i32(SPW)

    def zero(j, c):
        accv[pl.ds(j * _i32(16), 16)] = jnp.zeros((16,), _i32)
        return c

    lax.fori_loop(_i32(0), _i32(SPW // 16), zero, _i32(0))
    for c in range(W):
        pltpu.sync_copy(accv, p_hbm.at[_i32(c), pl.ds(cb, SPW)])
        pltpu.sync_copy(hist_hbm.at[_i32(c), pl.ds(cb, SPW)], tmpv)

        def add(j, cc):
            accv[pl.ds(j * _i32(16), 16)] = (accv[pl.ds(j * _i32(16), 16)]
                                       + tmpv[pl.ds(j * _i32(16), 16)])
            return cc

        lax.fori_loop(_i32(0), _i32(SPW // 16), add, _i32(0))
    pltpu.sync_copy(accv, deg_hbm.at[pl.ds(cb, SPW)])

    def tot(j, acc):
        return (acc + jnp.sum(accv[pl.ds(j * _i32(16), 16)], dtype=_i32)).astype(_i32)

    total = lax.fori_loop(_i32(0), _i32(SPW // 16), tot, _i32(0))
    sumv[...] = jnp.zeros((16,), _i32) + total
    pltpu.sync_copy(sumv, sums_hbm.at[w])


def _k3_rowptr(deg_hbm, sums_hbm, rp_hbm, degb, sumsv, rpb, sem):
    w = _wid()
    cb = w * _i32(SPW)
    pltpu.sync_copy(sums_hbm, sumsv)
    lanes = _iota16()
    zeros = jnp.zeros((16,), _i32)
    s_lo = plsc.load_gather(sumsv, [lanes, zeros])
    s_hi = plsc.load_gather(sumsv, [lanes + _i32(16), zeros])
    base = (jnp.sum(jnp.where(lanes < w, s_lo, jnp.zeros((16,), _i32)),
                    dtype=_i32)
            + jnp.sum(jnp.where(lanes + _i32(16) < w, s_hi,
                                jnp.zeros((16,), _i32)), dtype=_i32)).astype(_i32)
    pltpu.sync_copy(deg_hbm.at[pl.ds(cb, SPW)], degb)

    def vreg(j, carry):
        v = degb[pl.ds(j * _i32(16), 16)]
        cs = plsc.cumsum(v)
        rpb[pl.ds(j * _i32(16), 16)] = cs - v + carry
        return (carry + jnp.sum(v, dtype=_i32)).astype(_i32)

    lax.fori_loop(_i32(0), _i32(SPW // 16), vreg, base)
    pltpu.sync_copy(rpb, rp_hbm.at[pl.ds(cb, SPW)])


def _k4_scatter(row_hbm, col_hbm, p_hbm, rp_hbm, cs_hbm,
                ptrv, rowb, colb, posb, sem):
    w = _wid()

    def initblk(b, c):
        pltpu.sync_copy(p_hbm.at[w, pl.ds(b * _i32(BLK), BLK)], rowb)
        pltpu.sync_copy(rp_hbm.at[pl.ds(b * _i32(BLK), BLK)], colb)

        def add(j, cc):
            ptrv[pl.ds(b * _i32(BLK) + j * _i32(16), 16)] = (
                rowb[pl.ds(j * _i32(16), 16)] + colb[pl.ds(j * _i32(16), 16)])
            return cc

        lax.fori_loop(_i32(0), _i32(BLK // 16), add, _i32(0))
        return c

    lax.fori_loop(_i32(0), _i32(NP // BLK), initblk, _i32(0))

    start = w * _i32(CHW)
    ec = _edge_count(w)
    nblk = (ec + _i32(BLK - 1)) // _i32(BLK)

    def blk(b, c):
        pltpu.sync_copy(row_hbm.at[pl.ds(start + b * _i32(BLK), BLK)], rowb)
        pltpu.sync_copy(col_hbm.at[pl.ds(start + b * _i32(BLK), BLK)], colb)
        bs = jnp.minimum(_i32(BLK), ec - b * _i32(BLK))

        def rowloop(j, c2):
            for v in range(8):
                sl = pl.ds(j * _i32(128) + v * 16, 16)
                r = rowb[sl]
                cnt, last = plsc.scan_count(r)
                old = plsc.load_gather(ptrv, [r])
                posb[j, pl.ds(v * 16, 16)] = old + cnt - _i32(1)
                plsc.addupdate_scatter(ptrv, [r], cnt, mask=last)
            pltpu.async_copy(colb.at[pl.ds(j * _i32(128), 128)],
                             cs_hbm.at[posb.at[j]], sem)
            return c2

        nrows = bs // _i32(128)
        lax.fori_loop(_i32(0), nrows, rowloop, c)

        def drain(j, c2):
            pltpu.make_async_copy(colb.at[pl.ds(_i32(0), 128)],
                                  cs_hbm.at[posb.at[_i32(0)]], sem).wait()
            return c2

        lax.fori_loop(_i32(0), nrows, drain, c)
        return c

    lax.fori_loop(_i32(0), nblk, blk, _i32(0))


def _k5_walk(deg_hbm, rp_hbm, cs_hbm, u_hbm, w2_hbm, w3_hbm,
             curb, degb, rpb, adrb, nxtb, ub, sem):
    w = _wid()
    base = w * _i32(SPW)
    nrows = SPW // 128
    lanes = _iota16()

    def init(j, c):
        for v in range(8):
            curb[j, pl.ds(v * 16, 16)] = base + j * _i32(128) + _i32(v * 16) + lanes
        return c

    lax.fori_loop(_i32(0), _i32(nrows), init, _i32(0))

    for t in range(WALK_LEN):
        pltpu.sync_copy(u_hbm.at[pl.ds(_i32(t * NP) + base, SPW)], ub)
        hs = []
        for j in range(nrows):
            hs.append(pltpu.async_copy(deg_hbm.at[curb.at[_i32(j)]], degb.at[_i32(j)], sem))
            hs.append(pltpu.async_copy(rp_hbm.at[curb.at[_i32(j)]], rpb.at[_i32(j)], sem))
        for h in hs:
            h.wait()

        def addr(j, c):
            for v in range(8):
                sl = pl.ds(v * 16, 16)
                d = degb[j, sl]
                u = ub[pl.ds(j * _i32(128) + v * 16, 16)]
                idx = (u * d.astype(_f32)).astype(_i32)
                idx = jnp.minimum(idx, jnp.maximum(d - _i32(1), jnp.zeros((16,), _i32)))
                adrb[j, sl] = rpb[j, sl] + idx
            return c

        lax.fori_loop(_i32(0), _i32(nrows), addr, _i32(0))
        hs = [pltpu.async_copy(cs_hbm.at[adrb.at[_i32(j)]], nxtb.at[_i32(j)], sem)
              for j in range(nrows)]
        for h in hs:
            h.wait()

        def step(j, c):
            for v in range(8):
                sl = pl.ds(v * 16, 16)
                d = degb[j, sl]
                curb[j, sl] = jnp.where(d > 0, nxtb[j, sl], curb[j, sl])
            return c

        lax.fori_loop(_i32(0), _i32(nrows), step, _i32(0))
        if t >= 1:
            out = w2_hbm if t == 1 else w3_hbm
            for j in range(nrows):
                pltpu.sync_copy(curb.at[_i32(j)],
                                out.at[pl.ds(base + _i32(j * 128), 128)])


def _sc_params():
    return pltpu.CompilerParams(needs_layout_passes=False)


def _mesh():
    return plsc.VectorSubcoreMesh(core_axis_name="c", subcore_axis_name="s")


def kernel(edge_index, edge_weight):
    row32 = edge_index[0].astype(_i32)
    col32 = edge_index[1].astype(_i32)
    row_pad = jnp.pad(row32, (0, EPAD - E))
    col_pad = jnp.pad(col32, (0, EPAD - E))

    keys = jax.random.split(jax.random.key(42), WALK_LEN)
    u = jnp.stack([jax.random.uniform(k, (N,)).astype(_f32) for k in keys])
    u_pad = jnp.pad(u, ((0, 0), (0, NP - N))).reshape(-1)

    mesh = _mesh()

    hist = pl.kernel(
        _k1_hist,
        out_type=jax.ShapeDtypeStruct((W, NP), _i32),
        mesh=mesh,
        compiler_params=_sc_params(),
        scratch_types=[pltpu.VMEM((NP,), _i32),
                       pltpu.VMEM((BLK,), _i32),
                       pltpu.SemaphoreType.DMA],
    )(row_pad)

    p, deg, sums = pl.kernel(
        _k2_prefix,
        out_type=(jax.ShapeDtypeStruct((W, NP), _i32),
                  jax.ShapeDtypeStruct((NP,), _i32),
                  jax.ShapeDtypeStruct((W, 16), _i32)),
        mesh=mesh,
        compiler_params=_sc_params(),
        scratch_types=[pltpu.VMEM((SPW,), _i32),
                       pltpu.VMEM((SPW,), _i32),
                       pltpu.VMEM((16,), _i32),
                       pltpu.SemaphoreType.DMA],
    )(hist)

    rowptr = pl.kernel(
        _k3_rowptr,
        out_type=jax.ShapeDtypeStruct((NP,), _i32),
        mesh=mesh,
        compiler_params=_sc_params(),
        scratch_types=[pltpu.VMEM((SPW,), _i32),
                       pltpu.VMEM((W, 16), _i32),
                       pltpu.VMEM((SPW,), _i32),
                       pltpu.SemaphoreType.DMA],
    )(deg, sums)

    col_sorted = pl.kernel(
        _k4_scatter,
        out_type=jax.ShapeDtypeStruct((CSP,), _i32),
        mesh=mesh,
        compiler_params=_sc_params(),
        scratch_types=[pltpu.VMEM((NP,), _i32),
                       pltpu.VMEM((BLK,), _i32),
                       pltpu.VMEM((BLK,), _i32),
                       pltpu.VMEM((BLK // 128, 128), _i32),
                       pltpu.SemaphoreType.DMA],
    )(row_pad, col_pad, p, rowptr)

    w2, w3 = pl.kernel(
        _k5_walk,
        out_type=(jax.ShapeDtypeStruct((NP,), _i32),
                  jax.ShapeDtypeStruct((NP,), _i32)),
        mesh=mesh,
        compiler_params=_sc_params(),
        scratch_types=[pltpu.VMEM((SPW // 128, 128), _i32),
                       pltpu.VMEM((SPW // 128, 128), _i32),
                       pltpu.VMEM((SPW // 128, 128), _i32),
                       pltpu.VMEM((SPW // 128, 128), _i32),
                       pltpu.VMEM((SPW // 128, 128), _i32),
                       pltpu.VMEM((SPW,), _f32),
                       pltpu.SemaphoreType.DMA],
    )(deg, rowptr, col_sorted, u_pad)

    start = jnp.arange(N, dtype=edge_index.dtype)
    row_new = jnp.broadcast_to(start[:, None], (N, 2)).reshape(-1)
    col_new = jnp.stack([w2[:N], w3[:N]], axis=1).reshape(-1).astype(
        edge_index.dtype)
    edge_index_out = jnp.concatenate(
        [edge_index, jnp.stack([row_new, col_new])], axis=1)
    edge_weight_out = jnp.concatenate(
        [edge_weight, jnp.ones(2 * N, dtype=edge_weight.dtype)])
    return edge_index_out, edge_weight_out


# speedup vs baseline: 2.4314x; 2.4314x over previous
import functools

import jax
import jax.numpy as jnp
from jax import lax
from jax.experimental import pallas as pl
from jax.experimental.pallas import tpu as pltpu
from jax.experimental.pallas import tpu_sc as plsc

N = 100000
E = 3200000
W = 32
NP = 102400
SPW = NP // W
CHW = 100096
ECL = E - 31 * CHW
BLK = 4096
EPAD = E + BLK
CSP = E + 512
WALK_LEN = 3
NSEG = E // 2
SINK = 64
PCH = 102400

_i32 = jnp.int32
_f32 = jnp.float32


def _wid():
    return (lax.axis_index("s") * _i32(2) + lax.axis_index("c")).astype(_i32)


def _iota16():
    return lax.iota(_i32, 16)


def _edge_count(w):
    return jnp.where(w == _i32(W - 1), _i32(ECL), _i32(CHW)).astype(_i32)


def _k1_hist(row_hbm, hist_hbm, histv, rowb, sem):
    w = _wid()

    def zero(j, c):
        histv[pl.ds(j * _i32(16), 16)] = jnp.zeros((16,), _i32)
        return c

    lax.fori_loop(_i32(0), _i32(NP // 16), zero, _i32(0))
    start = w * _i32(CHW)
    ec = _edge_count(w)
    nblk = (ec + _i32(BLK - 1)) // _i32(BLK)

    def blk(b, c):
        pltpu.sync_copy(row_hbm.at[pl.ds(start + b * _i32(BLK), BLK)], rowb)
        bs = jnp.minimum(_i32(BLK), ec - b * _i32(BLK))

        def vreg(j, c2):
            r = rowb[pl.ds(j * _i32(16), 16)]
            cnt, last = plsc.scan_count(r)
            plsc.addupdate_scatter(histv, [r], cnt, mask=last)
            return c2

        lax.fori_loop(_i32(0), bs // _i32(16), vreg, c)
        return c

    lax.fori_loop(_i32(0), nblk, blk, _i32(0))
    pltpu.sync_copy(histv, hist_hbm.at[w])


def _k2_prefix(hist_hbm, p_hbm, deg_hbm, sums_hbm, accv, tmpv, sumv, sem):
    w = _wid()
    cb = w * _i32(SPW)

    def zero(j, c):
        accv[pl.ds(j * _i32(16), 16)] = jnp.zeros((16,), _i32)
        return c

    lax.fori_loop(_i32(0), _i32(SPW // 16), zero, _i32(0))
    for c in range(W):
        pltpu.sync_copy(accv, p_hbm.at[_i32(c), pl.ds(cb, SPW)])
        pltpu.sync_copy(hist_hbm.at[_i32(c), pl.ds(cb, SPW)], tmpv)

        def add(j, cc):
            accv[pl.ds(j * _i32(16), 16)] = (accv[pl.ds(j * _i32(16), 16)]
                                       + tmpv[pl.ds(j * _i32(16), 16)])
            return cc

        lax.fori_loop(_i32(0), _i32(SPW // 16), add, _i32(0))
    pltpu.sync_copy(accv, deg_hbm.at[pl.ds(cb, SPW)])

    def tot(j, acc):
        return (acc + jnp.sum(accv[pl.ds(j * _i32(16), 16)], dtype=_i32)).astype(_i32)

    total = lax.fori_loop(_i32(0), _i32(SPW // 16), tot, _i32(0))
    sumv[...] = jnp.zeros((16,), _i32) + total
    pltpu.sync_copy(sumv, sums_hbm.at[w])


def _k4a_positions(row_hbm, p_hbm, deg_hbm, pos_hbm, rp_hbm,
                   ptrv, rowb, colb, posb, sem):
    w = _wid()

    def initblk(b, carry):
        pltpu.sync_copy(p_hbm.at[w, pl.ds(b * _i32(BLK), BLK)], rowb)
        pltpu.sync_copy(deg_hbm.at[pl.ds(b * _i32(BLK), BLK)], colb)

        def add(j, cin):
            v = colb[pl.ds(j * _i32(16), 16)]
            excl = plsc.cumsum(v) - v + cin
            ptrv[pl.ds(b * _i32(BLK) + j * _i32(16), 16)] = (
                rowb[pl.ds(j * _i32(16), 16)] + excl)
            return (cin + jnp.sum(v, dtype=_i32)).astype(_i32)

        return lax.fori_loop(_i32(0), _i32(BLK // 16), add, carry)

    lax.fori_loop(_i32(0), _i32(NP // BLK), initblk, _i32(0))

    @pl.when(w == _i32(0))
    def _():
        pltpu.sync_copy(ptrv, rp_hbm)

    start = w * _i32(CHW)
    ec = _edge_count(w)
    nblk = (ec + _i32(BLK - 1)) // _i32(BLK)

    def blk(b, c):
        pltpu.sync_copy(row_hbm.at[pl.ds(start + b * _i32(BLK), BLK)], rowb)
        bs = jnp.minimum(_i32(BLK), ec - b * _i32(BLK))

        def rowloop(j, c2):
            for v in range(8):
                sl = pl.ds(j * _i32(128) + v * 16, 16)
                r = rowb[sl]
                cnt, last = plsc.scan_count(r)
                old = plsc.load_gather(ptrv, [r])
                posb[sl] = old + cnt - _i32(1)
                plsc.addupdate_scatter(ptrv, [r], cnt, mask=last)
            return c2

        lax.fori_loop(_i32(0), bs // _i32(128), rowloop, c)
        pltpu.sync_copy(posb,
                        pos_hbm.at[pl.ds(w * _i32(PCH) + b * _i32(BLK), BLK)])
        return c

    lax.fori_loop(_i32(0), nblk, blk, _i32(0))


def _k4b_apply(pos_hbm, col_hbm, cs_hbm, posin, colin, posb, spm, sem):
    c_ax = lax.axis_index("c").astype(_i32)
    s_ax = lax.axis_index("s").astype(_i32)
    rlo = c_ax * _i32(NSEG)

    for half in range(2):
        w = s_ax * _i32(2) + _i32(half)
        pstart = w * _i32(PCH)
        cstart = w * _i32(CHW)
        ec = _edge_count(w)
        nblk = (ec + _i32(BLK - 1)) // _i32(BLK)

        def blk(b, c):
            pltpu.sync_copy(pos_hbm.at[pl.ds(pstart + b * _i32(BLK), BLK)],
                            posin)
            pltpu.sync_copy(col_hbm.at[pl.ds(cstart + b * _i32(BLK), BLK)],
                            colin)
            bs = jnp.minimum(_i32(BLK), ec - b * _i32(BLK))

            def rowloop(j, c2):
                for v in range(8):
                    p = posin[pl.ds(j * _i32(128) + v * 16, 16)]
                    loc = p - rlo
                    ok = (loc >= _i32(0)) & (loc < _i32(NSEG))
                    sink = _i32(NSEG) + (p & _i32(SINK - 1))
                    posb[j, pl.ds(v * 16, 16)] = jnp.where(ok, loc, sink)
                pltpu.async_copy(colin.at[pl.ds(j * _i32(128), 128)],
                                 spm.at[posb.at[j]], sem)
                return c2

            nrows = bs // _i32(128)
            lax.fori_loop(_i32(0), nrows, rowloop, c)

            def drain(j, c2):
                pltpu.make_async_copy(colin.at[pl.ds(_i32(0), 128)],
                                      spm.at[posb.at[_i32(0)]], sem).wait()
                return c2

            lax.fori_loop(_i32(0), nrows, drain, c)
            return c

        lax.fori_loop(_i32(0), nblk, blk, _i32(0))

    plsc.subcore_barrier()
    seg = NSEG // 16
    off = s_ax * _i32(seg)

    def out(k, c):
        o = off + k * _i32(4000)
        pltpu.sync_copy(spm.at[pl.ds(o, 4000)], colin.at[pl.ds(0, 4000)])
        pltpu.sync_copy(colin.at[pl.ds(0, 4000)],
                        cs_hbm.at[pl.ds(rlo + o, 4000)])
        return c

    lax.fori_loop(_i32(0), _i32(seg // 4000), out, _i32(0))


def _k5_walk(deg_hbm, rp_hbm, cs_hbm, u_hbm, w2_hbm, w3_hbm,
             curb, degb, rpb, adrb, nxtb, ub, sem):
    w = _wid()
    base = w * _i32(SPW)
    nrows = SPW // 128
    lanes = _iota16()

    def init(j, c):
        for v in range(8):
            curb[j, pl.ds(v * 16, 16)] = base + j * _i32(128) + _i32(v * 16) + lanes
        return c

    lax.fori_loop(_i32(0), _i32(nrows), init, _i32(0))

    for t in range(WALK_LEN):
        pltpu.sync_copy(u_hbm.at[pl.ds(_i32(t * NP) + base, SPW)], ub)
        hs = []
        for j in range(nrows):
            hs.append(pltpu.async_copy(deg_hbm.at[curb.at[_i32(j)]], degb.at[_i32(j)], sem))
            hs.append(pltpu.async_copy(rp_hbm.at[curb.at[_i32(j)]], rpb.at[_i32(j)], sem))
        for h in hs:
            h.wait()

        def addr(j, c):
            for v in range(8):
                sl = pl.ds(v * 16, 16)
                d = degb[j, sl]
                u = ub[pl.ds(j * _i32(128) + v * 16, 16)]
                idx = (u * d.astype(_f32)).astype(_i32)
                idx = jnp.minimum(idx, jnp.maximum(d - _i32(1), jnp.zeros((16,), _i32)))
                adrb[j, sl] = rpb[j, sl] + idx
            return c

        lax.fori_loop(_i32(0), _i32(nrows), addr, _i32(0))
        hs = [pltpu.async_copy(cs_hbm.at[adrb.at[_i32(j)]], nxtb.at[_i32(j)], sem)
              for j in range(nrows)]
        for h in hs:
            h.wait()

        def step(j, c):
            for v in range(8):
                sl = pl.ds(v * 16, 16)
                d = degb[j, sl]
                curb[j, sl] = jnp.where(d > 0, nxtb[j, sl], curb[j, sl])
            return c

        lax.fori_loop(_i32(0), _i32(nrows), step, _i32(0))
        if t >= 1:
            out = w2_hbm if t == 1 else w3_hbm
            for j in range(nrows):
                pltpu.sync_copy(curb.at[_i32(j)],
                                out.at[pl.ds(base + _i32(j * 128), 128)])


def _sc_params():
    return pltpu.CompilerParams(needs_layout_passes=False)


def _mesh():
    return plsc.VectorSubcoreMesh(core_axis_name="c", subcore_axis_name="s")


def kernel(edge_index, edge_weight):
    row32 = edge_index[0].astype(_i32)
    col32 = edge_index[1].astype(_i32)
    row_pad = jnp.pad(row32, (0, EPAD - E))
    col_pad = jnp.pad(col32, (0, EPAD - E))

    keys = jax.random.split(jax.random.key(42), WALK_LEN)
    u = jnp.stack([jax.random.uniform(k, (N,)).astype(_f32) for k in keys])
    u_pad = jnp.pad(u, ((0, 0), (0, NP - N))).reshape(-1)

    mesh = _mesh()

    hist = pl.kernel(
        _k1_hist,
        out_type=jax.ShapeDtypeStruct((W, NP), _i32),
        mesh=mesh,
        compiler_params=_sc_params(),
        scratch_types=[pltpu.VMEM((NP,), _i32),
                       pltpu.VMEM((BLK,), _i32),
                       pltpu.SemaphoreType.DMA],
    )(row_pad)

    p, deg, sums = pl.kernel(
        _k2_prefix,
        out_type=(jax.ShapeDtypeStruct((W, NP), _i32),
                  jax.ShapeDtypeStruct((NP,), _i32),
                  jax.ShapeDtypeStruct((W, 16), _i32)),
        mesh=mesh,
        compiler_params=_sc_params(),
        scratch_types=[pltpu.VMEM((SPW,), _i32),
                       pltpu.VMEM((SPW,), _i32),
                       pltpu.VMEM((16,), _i32),
                       pltpu.SemaphoreType.DMA],
    )(hist)

    pos, rowptr = pl.kernel(
        _k4a_positions,
        out_type=(jax.ShapeDtypeStruct((W * PCH,), _i32),
                  jax.ShapeDtypeStruct((NP,), _i32)),
        mesh=mesh,
        compiler_params=_sc_params(),
        scratch_types=[pltpu.VMEM((NP,), _i32),
                       pltpu.VMEM((BLK,), _i32),
                       pltpu.VMEM((BLK,), _i32),
                       pltpu.VMEM((BLK,), _i32),
                       pltpu.SemaphoreType.DMA],
    )(row_pad, p, deg)

    col_sorted = pl.kernel(
        _k4b_apply,
        out_type=jax.ShapeDtypeStruct((CSP,), _i32),
        mesh=mesh,
        compiler_params=_sc_params(),
        scratch_types=[pltpu.VMEM((BLK,), _i32),
                       pltpu.VMEM((BLK,), _i32),
                       pltpu.VMEM((BLK // 128, 128), _i32),
                       pltpu.VMEM_SHARED((NSEG + SINK,), _i32),
                       pltpu.SemaphoreType.DMA],
    )(pos, col_pad)

    w2, w3 = pl.kernel(
        _k5_walk,
        out_type=(jax.ShapeDtypeStruct((NP,), _i32),
                  jax.ShapeDtypeStruct((NP,), _i32)),
        mesh=mesh,
        compiler_params=_sc_params(),
        scratch_types=[pltpu.VMEM((SPW // 128, 128), _i32),
                       pltpu.VMEM((SPW // 128, 128), _i32),
                       pltpu.VMEM((SPW // 128, 128), _i32),
                       pltpu.VMEM((SPW // 128, 128), _i32),
                       pltpu.VMEM((SPW // 128, 128), _i32),
                       pltpu.VMEM((SPW,), _f32),
                       pltpu.SemaphoreType.DMA],
    )(deg, rowptr, col_sorted, u_pad)

    start = jnp.arange(N, dtype=edge_index.dtype)
    row_new = jnp.broadcast_to(start[:, None], (N, 2)).reshape(-1)
    col_new = jnp.stack([w2[:N], w3[:N]], axis=1).reshape(-1).astype(
        edge_index.dtype)
    edge_index_out = jnp.concatenate(
        [edge_index, jnp.stack([row_new, col_new])], axis=1)
    edge_weight_out = jnp.concatenate(
        [edge_weight, jnp.ones(2 * N, dtype=edge_weight.dtype)])
    return edge_index_out, edge_weight_out


# speedup vs baseline: 2.4328x; 1.0006x over previous
import functools

import jax
import jax.numpy as jnp
from jax import lax
from jax.experimental import pallas as pl
from jax.experimental.pallas import tpu as pltpu
from jax.experimental.pallas import tpu_sc as plsc

N = 100000
E = 3200000
W = 32
NP = 102400
SPW = NP // W
CHW = 100096
ECL = E - 31 * CHW
BLK = 4096
EPAD = E + BLK
CSP = E + 512
WALK_LEN = 3
NSEG = E // 2
SINK = 64
PCH = 102400

_i32 = jnp.int32
_f32 = jnp.float32


def _wid():
    return (lax.axis_index("s") * _i32(2) + lax.axis_index("c")).astype(_i32)


def _iota16():
    return lax.iota(_i32, 16)


def _edge_count(w):
    return jnp.where(w == _i32(W - 1), _i32(ECL), _i32(CHW)).astype(_i32)


def _k1_hist(row_hbm, hist_hbm, histv, rowb, sem):
    w = _wid()

    def zero(j, c):
        histv[pl.ds(j * _i32(16), 16)] = jnp.zeros((16,), _i32)
        return c

    lax.fori_loop(_i32(0), _i32(NP // 16), zero, _i32(0))
    start = w * _i32(CHW)
    ec = _edge_count(w)
    nblk = (ec + _i32(BLK - 1)) // _i32(BLK)

    def blk(b, c):
        pltpu.sync_copy(row_hbm.at[pl.ds(start + b * _i32(BLK), BLK)], rowb)
        bs = jnp.minimum(_i32(BLK), ec - b * _i32(BLK))

        def rowloop(j, c2):
            for v in range(8):
                r = rowb[pl.ds(j * _i32(128) + v * 16, 16)]
                cnt, last = plsc.scan_count(r)
                plsc.addupdate_scatter(histv, [r], cnt, mask=last)
            return c2

        lax.fori_loop(_i32(0), bs // _i32(128), rowloop, c)
        return c

    lax.fori_loop(_i32(0), nblk, blk, _i32(0))
    pltpu.sync_copy(histv, hist_hbm.at[w])


def _k2_prefix(hist_hbm, p_hbm, deg_hbm, accv, tmpv, sem):
    w = _wid()
    cb = w * _i32(SPW)

    def zero(j, c):
        accv[pl.ds(j * _i32(16), 16)] = jnp.zeros((16,), _i32)
        return c

    lax.fori_loop(_i32(0), _i32(SPW // 16), zero, _i32(0))
    for c in range(W):
        pltpu.sync_copy(accv, p_hbm.at[_i32(c), pl.ds(cb, SPW)])
        pltpu.sync_copy(hist_hbm.at[_i32(c), pl.ds(cb, SPW)], tmpv)

        def add(j, cc):
            accv[pl.ds(j * _i32(16), 16)] = (accv[pl.ds(j * _i32(16), 16)]
                                       + tmpv[pl.ds(j * _i32(16), 16)])
            return cc

        lax.fori_loop(_i32(0), _i32(SPW // 16), add, _i32(0))
    pltpu.sync_copy(accv, deg_hbm.at[pl.ds(cb, SPW)])


def _k4a_positions(row_hbm, p_hbm, deg_hbm, pos_hbm, rp_hbm,
                   ptrv, rowb, colb, posb, sem):
    w = _wid()

    def initblk(b, carry):
        pltpu.sync_copy(p_hbm.at[w, pl.ds(b * _i32(BLK), BLK)], rowb)
        pltpu.sync_copy(deg_hbm.at[pl.ds(b * _i32(BLK), BLK)], colb)

        def add(j, cin):
            v = colb[pl.ds(j * _i32(16), 16)]
            excl = plsc.cumsum(v) - v + cin
            ptrv[pl.ds(b * _i32(BLK) + j * _i32(16), 16)] = (
                rowb[pl.ds(j * _i32(16), 16)] + excl)
            return (cin + jnp.sum(v, dtype=_i32)).astype(_i32)

        return lax.fori_loop(_i32(0), _i32(BLK // 16), add, carry)

    lax.fori_loop(_i32(0), _i32(NP // BLK), initblk, _i32(0))

    @pl.when(w == _i32(0))
    def _():
        pltpu.sync_copy(ptrv, rp_hbm)

    start = w * _i32(CHW)
    ec = _edge_count(w)
    nblk = (ec + _i32(BLK - 1)) // _i32(BLK)

    def blk(b, c):
        pltpu.sync_copy(row_hbm.at[pl.ds(start + b * _i32(BLK), BLK)], rowb)
        bs = jnp.minimum(_i32(BLK), ec - b * _i32(BLK))

        def rowloop(j, c2):
            for v in range(8):
                sl = pl.ds(j * _i32(128) + v * 16, 16)
                r = rowb[sl]
                cnt, last = plsc.scan_count(r)
                old = plsc.load_gather(ptrv, [r])
                posb[sl] = old + cnt - _i32(1)
                plsc.addupdate_scatter(ptrv, [r], cnt, mask=last)
            return c2

        lax.fori_loop(_i32(0), bs // _i32(128), rowloop, c)
        pltpu.sync_copy(posb,
                        pos_hbm.at[pl.ds(w * _i32(PCH) + b * _i32(BLK), BLK)])
        return c

    lax.fori_loop(_i32(0), nblk, blk, _i32(0))


def _k4b_apply(pos_hbm, col_hbm, cs_hbm, posin, colin, posb, spm, sem):
    c_ax = lax.axis_index("c").astype(_i32)
    s_ax = lax.axis_index("s").astype(_i32)
    rlo = c_ax * _i32(NSEG)

    for half in range(2):
        w = s_ax * _i32(2) + _i32(half)
        pstart = w * _i32(PCH)
        cstart = w * _i32(CHW)
        ec = _edge_count(w)
        nblk = (ec + _i32(BLK - 1)) // _i32(BLK)

        def blk(b, c):
            pltpu.sync_copy(pos_hbm.at[pl.ds(pstart + b * _i32(BLK), BLK)],
                            posin)
            pltpu.sync_copy(col_hbm.at[pl.ds(cstart + b * _i32(BLK), BLK)],
                            colin)
            bs = jnp.minimum(_i32(BLK), ec - b * _i32(BLK))

            def rowloop(j, c2):
                for v in range(8):
                    p = posin[pl.ds(j * _i32(128) + v * 16, 16)]
                    loc = p - rlo
                    ok = (loc >= _i32(0)) & (loc < _i32(NSEG))
                    sink = _i32(NSEG) + (p & _i32(SINK - 1))
                    posb[j, pl.ds(v * 16, 16)] = jnp.where(ok, loc, sink)
                pltpu.async_copy(colin.at[pl.ds(j * _i32(128), 128)],
                                 spm.at[posb.at[j]], sem)
                return c2

            nrows = bs // _i32(128)
            lax.fori_loop(_i32(0), nrows, rowloop, c)

            def drain(j, c2):
                pltpu.make_async_copy(colin.at[pl.ds(_i32(0), 128)],
                                      spm.at[posb.at[_i32(0)]], sem).wait()
                return c2

            lax.fori_loop(_i32(0), nrows, drain, c)
            return c

        lax.fori_loop(_i32(0), nblk, blk, _i32(0))

    plsc.subcore_barrier()
    seg = NSEG // 16
    off = s_ax * _i32(seg)

    def out(k, c):
        o = off + k * _i32(4000)
        pltpu.sync_copy(spm.at[pl.ds(o, 4000)], colin.at[pl.ds(0, 4000)])
        pltpu.sync_copy(colin.at[pl.ds(0, 4000)],
                        cs_hbm.at[pl.ds(rlo + o, 4000)])
        return c

    lax.fori_loop(_i32(0), _i32(seg // 4000), out, _i32(0))


def _k5_walk(deg_hbm, rp_hbm, cs_hbm, u_hbm, w2_hbm, w3_hbm,
             curb, degb, rpb, adrb, nxtb, ub, sem):
    w = _wid()
    base = w * _i32(SPW)
    nrows = SPW // 128
    lanes = _iota16()

    def init(j, c):
        for v in range(8):
            curb[j, pl.ds(v * 16, 16)] = base + j * _i32(128) + _i32(v * 16) + lanes
        return c

    lax.fori_loop(_i32(0), _i32(nrows), init, _i32(0))

    for t in range(WALK_LEN):
        pltpu.sync_copy(u_hbm.at[pl.ds(_i32(t * NP) + base, SPW)], ub)
        hs = []
        for j in range(nrows):
            hs.append(pltpu.async_copy(deg_hbm.at[curb.at[_i32(j)]], degb.at[_i32(j)], sem))
            hs.append(pltpu.async_copy(rp_hbm.at[curb.at[_i32(j)]], rpb.at[_i32(j)], sem))
        for h in hs:
            h.wait()

        def addr(j, c):
            for v in range(8):
                sl = pl.ds(v * 16, 16)
                d = degb[j, sl]
                u = ub[pl.ds(j * _i32(128) + v * 16, 16)]
                idx = (u * d.astype(_f32)).astype(_i32)
                idx = jnp.minimum(idx, jnp.maximum(d - _i32(1), jnp.zeros((16,), _i32)))
                adrb[j, sl] = rpb[j, sl] + idx
            return c

        lax.fori_loop(_i32(0), _i32(nrows), addr, _i32(0))
        hs = [pltpu.async_copy(cs_hbm.at[adrb.at[_i32(j)]], nxtb.at[_i32(j)], sem)
              for j in range(nrows)]
        for h in hs:
            h.wait()

        def step(j, c):
            for v in range(8):
                sl = pl.ds(v * 16, 16)
                d = degb[j, sl]
                curb[j, sl] = jnp.where(d > 0, nxtb[j, sl], curb[j, sl])
            return c

        lax.fori_loop(_i32(0), _i32(nrows), step, _i32(0))
        if t >= 1:
            out = w2_hbm if t == 1 else w3_hbm
            for j in range(nrows):
                pltpu.sync_copy(curb.at[_i32(j)],
                                out.at[pl.ds(base + _i32(j * 128), 128)])


def _sc_params():
    return pltpu.CompilerParams(needs_layout_passes=False)


def _mesh():
    return plsc.VectorSubcoreMesh(core_axis_name="c", subcore_axis_name="s")


def kernel(edge_index, edge_weight):
    row32 = edge_index[0].astype(_i32)
    col32 = edge_index[1].astype(_i32)
    row_pad = jnp.pad(row32, (0, EPAD - E))
    col_pad = jnp.pad(col32, (0, EPAD - E))

    keys = jax.random.split(jax.random.key(42), WALK_LEN)
    u = jnp.stack([jax.random.uniform(k, (N,)).astype(_f32) for k in keys])
    u_pad = jnp.pad(u, ((0, 0), (0, NP - N))).reshape(-1)

    mesh = _mesh()

    hist = pl.kernel(
        _k1_hist,
        out_type=jax.ShapeDtypeStruct((W, NP), _i32),
        mesh=mesh,
        compiler_params=_sc_params(),
        scratch_types=[pltpu.VMEM((NP,), _i32),
                       pltpu.VMEM((BLK,), _i32),
                       pltpu.SemaphoreType.DMA],
    )(row_pad)

    p, deg = pl.kernel(
        _k2_prefix,
        out_type=(jax.ShapeDtypeStruct((W, NP), _i32),
                  jax.ShapeDtypeStruct((NP,), _i32)),
        mesh=mesh,
        compiler_params=_sc_params(),
        scratch_types=[pltpu.VMEM((SPW,), _i32),
                       pltpu.VMEM((SPW,), _i32),
                       pltpu.SemaphoreType.DMA],
    )(hist)

    pos, rowptr = pl.kernel(
        _k4a_positions,
        out_type=(jax.ShapeDtypeStruct((W * PCH,), _i32),
                  jax.ShapeDtypeStruct((NP,), _i32)),
        mesh=mesh,
        compiler_params=_sc_params(),
        scratch_types=[pltpu.VMEM((NP,), _i32),
                       pltpu.VMEM((BLK,), _i32),
                       pltpu.VMEM((BLK,), _i32),
                       pltpu.VMEM((BLK,), _i32),
                       pltpu.SemaphoreType.DMA],
    )(row_pad, p, deg)

    col_sorted = pl.kernel(
        _k4b_apply,
        out_type=jax.ShapeDtypeStruct((CSP,), _i32),
        mesh=mesh,
        compiler_params=_sc_params(),
        scratch_types=[pltpu.VMEM((BLK,), _i32),
                       pltpu.VMEM((BLK,), _i32),
                       pltpu.VMEM((BLK // 128, 128), _i32),
                       pltpu.VMEM_SHARED((NSEG + SINK,), _i32),
                       pltpu.SemaphoreType.DMA],
    )(pos, col_pad)

    w2, w3 = pl.kernel(
        _k5_walk,
        out_type=(jax.ShapeDtypeStruct((NP,), _i32),
                  jax.ShapeDtypeStruct((NP,), _i32)),
        mesh=mesh,
        compiler_params=_sc_params(),
        scratch_types=[pltpu.VMEM((SPW // 128, 128), _i32),
                       pltpu.VMEM((SPW // 128, 128), _i32),
                       pltpu.VMEM((SPW // 128, 128), _i32),
                       pltpu.VMEM((SPW // 128, 128), _i32),
                       pltpu.VMEM((SPW // 128, 128), _i32),
                       pltpu.VMEM((SPW,), _f32),
                       pltpu.SemaphoreType.DMA],
    )(deg, rowptr, col_sorted, u_pad)

    start = jnp.arange(N, dtype=edge_index.dtype)
    row_new = jnp.broadcast_to(start[:, None], (N, 2)).reshape(-1)
    col_new = jnp.stack([w2[:N], w3[:N]], axis=1).reshape(-1).astype(
        edge_index.dtype)
    edge_index_out = jnp.concatenate(
        [edge_index, jnp.stack([row_new, col_new])], axis=1)
    edge_weight_out = jnp.concatenate(
        [edge_weight, jnp.ones(2 * N, dtype=edge_weight.dtype)])
    return edge_index_out, edge_weight_out
